# Initial kernel scaffold; baseline (speedup 1.0000x reference)
import jax
import jax.numpy as jnp
from jax.experimental import pallas as pl


def kernel(rays_o, rays_d, density_grid, k0_grid):
    raise NotImplementedError("write your pallas kernel here")


# speedup vs baseline: 735.0720x; 735.0720x over previous
import functools

import jax
import jax.numpy as jnp
import numpy as np
from jax import lax
from jax.experimental import pallas as pl
from jax.experimental.pallas import tpu as pltpu
from jax.experimental.pallas import tpu_sc as plsc

N_RAYS = 8192
XYZ_MIN = -1.0
XYZ_MAX = 1.0
NEAR = 0.2
FAR = 3.0
NUM_VOXELS = 1024000
VOXEL_SIZE = float(8.0 / NUM_VOXELS) ** (1.0 / 3.0) + 1e-6
G = int((XYZ_MAX - XYZ_MIN) / VOXEL_SIZE)
GRID_N = G * G * G
STEPSIZE = 0.5
STEPDIST = STEPSIZE * VOXEL_SIZE
INTERVAL = STEPSIZE
ALPHA_INIT = 1e-6
ACT_SHIFT = float(np.log(1.0 / (1.0 - ALPHA_INIT) - 1.0))
MAX_STEPS = int(np.ceil(float(np.sqrt(3.0) * (XYZ_MAX - XYZ_MIN)) / STEPDIST)) + 2

CS = 32
NSTREAM = CS * 32 // 128
NCHUNK = MAX_STEPS // CS
assert CS * NCHUNK == MAX_STEPS


def _rsqrt(x):
    i = lax.bitcast_convert_type(x, jnp.int32)
    y = lax.bitcast_convert_type(jnp.int32(0x5F3759DF) - (i >> 1), jnp.float32)
    for _ in range(3):
        y = y * (1.5 - 0.5 * x * y * y)
    return y


def _lerp(a, b, t):
    return a + t * (b - a)


@functools.partial(jax.jit, static_argnums=())
def _sc_render(table, prep):
    info = plsc.get_sparse_core_info()
    nc, ns = info.num_cores, info.num_subcores
    nw = nc * ns
    rpw = N_RAYS // nw
    ngroups = rpw // 16
    mesh = plsc.VectorSubcoreMesh(core_axis_name="c", subcore_axis_name="s")

    @functools.partial(
        pl.kernel,
        out_type=jax.ShapeDtypeStruct((N_RAYS, 3), jnp.float32),
        mesh=mesh,
        compiler_params=pltpu.CompilerParams(
            needs_layout_passes=False, use_tc_tiling_on_sc=False),
        scratch_types=[
            pltpu.VMEM((8, rpw), jnp.float32),
            pltpu.VMEM((NSTREAM, 128), jnp.int32),
            pltpu.VMEM((NSTREAM, 128, 16), jnp.float32),
            pltpu.VMEM((4, CS * 16), jnp.float32),
            pltpu.VMEM((rpw, 3), jnp.float32),
            pltpu.SemaphoreType.DMA,
        ],
    )
    def body(table_hbm, prep_hbm, out_hbm, prep_v, idx_v, rows_v, frac_v, outb_v, sem):
        wid = lax.axis_index("s") * nc + lax.axis_index("c")
        pltpu.sync_copy(prep_hbm.at[wid], prep_v)
        iota = lax.iota(jnp.int32, 16)
        zeros = jnp.zeros((16,), jnp.float32)
        ones = jnp.ones((16,), jnp.float32)

        def group_body(g, _):
            g16 = g * 16
            sx = prep_v[0, pl.ds(g16, 16)]
            sy = prep_v[1, pl.ds(g16, 16)]
            sz = prep_v[2, pl.ds(g16, 16)]
            ux = prep_v[3, pl.ds(g16, 16)]
            uy = prep_v[4, pl.ds(g16, 16)]
            uz = prep_v[5, pl.ds(g16, 16)]
            nst = prep_v[6, pl.ds(g16, 16)]
            nchunks = (jnp.max(nst).astype(jnp.int32) + (CS - 1)) // CS

            def chunk_body(c, carry):
                T, ar, ag, ab, aa = carry
                s0f = (c * CS).astype(jnp.float32)
                for j in range(CS):
                    distf = (s0f + float(j)) * STEPDIST
                    px = sx + ux * distf
                    py = sy + uy * distf
                    pz = sz + uz * distf
                    mo = ((jnp.abs(px) > 1.0) | (jnp.abs(py) > 1.0)
                          | (jnp.abs(pz) > 1.0))

                    def axis_ind(p):
                        ind = (p - XYZ_MIN) / (XYZ_MAX - XYZ_MIN) * float(G - 1)
                        ind = jnp.clip(ind, 0.0, float(G - 1))
                        i0 = jnp.minimum(ind.astype(jnp.int32), G - 2)
                        return i0, ind - i0.astype(jnp.float32)

                    xi, fx = axis_ind(px)
                    yi, fy = axis_ind(py)
                    zi, fz = axis_ind(pz)
                    basei = xi * (G * G) + yi * G + zi
                    sj, off = divmod(j * 32, 128)
                    idx_v[sj, pl.ds(off, 16)] = basei
                    idx_v[sj, pl.ds(off + 16, 16)] = basei + G * G
                    frac_v[0, pl.ds(j * 16, 16)] = fx
                    frac_v[1, pl.ds(j * 16, 16)] = fy
                    frac_v[2, pl.ds(j * 16, 16)] = fz
                    frac_v[3, pl.ds(j * 16, 16)] = jnp.where(mo, ones, zeros)
                copies = [
                    pltpu.async_copy(table_hbm.at[idx_v.at[t]], rows_v.at[t], sem)
                    for t in range(NSTREAM)
                ]
                for cp in copies:
                    cp.wait()
                for j in range(CS):
                    fx = frac_v[0, pl.ds(j * 16, 16)]
                    fy = frac_v[1, pl.ds(j * 16, 16)]
                    fz = frac_v[2, pl.ds(j * 16, 16)]
                    mof = frac_v[3, pl.ds(j * 16, 16)]
                    stepf = s0f + float(j)
                    live = (stepf < nst) & (mof < 0.5)
                    vals = []
                    for dxc in (0, 1):
                        sjj, p0 = divmod(j * 32 + dxc * 16, 128)
                        pv = iota + p0
                        rr = rows_v.at[sjj]
                        vals.append([
                            plsc.load_gather(
                                rr, [pv, jnp.full((16,), col, jnp.int32)])
                            for col in range(16)
                        ])
                    chan = []
                    for ch in range(4):
                        vy = []
                        for dxc in (0, 1):
                            vz0 = _lerp(vals[dxc][0 * 4 + ch],
                                        vals[dxc][1 * 4 + ch], fz)
                            vz1 = _lerp(vals[dxc][2 * 4 + ch],
                                        vals[dxc][3 * 4 + ch], fz)
                            vy.append(_lerp(vz0, vz1, fy))
                        chan.append(_lerp(vy[0], vy[1], fx))
                    e = jnp.exp(chan[0] + ACT_SHIFT)
                    alpha = 1.0 - _rsqrt(1.0 + e)
                    alpha = jnp.where(live, alpha, zeros)
                    w = alpha * T
                    T = T * (1.0 - jnp.minimum(alpha, 1.0 - 1e-6))
                    ar = ar + w / (1.0 + jnp.exp(-chan[1]))
                    ag = ag + w / (1.0 + jnp.exp(-chan[2]))
                    ab = ab + w / (1.0 + jnp.exp(-chan[3]))
                    aa = aa + w
                return (T, ar, ag, ab, aa)

            T, ar, ag, ab, aa = lax.fori_loop(
                0, nchunks, chunk_body, (ones, zeros, zeros, zeros, zeros))
            ridx = g16 + iota
            bg = 1.0 - aa
            plsc.store_scatter(outb_v, [ridx, jnp.full((16,), 0, jnp.int32)],
                               ar + bg)
            plsc.store_scatter(outb_v, [ridx, jnp.full((16,), 1, jnp.int32)],
                               ag + bg)
            plsc.store_scatter(outb_v, [ridx, jnp.full((16,), 2, jnp.int32)],
                               ab + bg)
            return 0

        lax.fori_loop(0, ngroups, group_body, 0)
        pltpu.sync_copy(outb_v, out_hbm.at[pl.ds(wid * rpw, rpw)])

    return body(table, prep)


def kernel(rays_o, rays_d, density_grid, k0_grid):
    avoid = jnp.maximum(jnp.abs(rays_d), 1e-6)
    t1 = (XYZ_MIN - rays_o) / avoid
    t2 = (XYZ_MAX - rays_o) / avoid
    t_min = jnp.clip(jnp.max(jnp.minimum(t1, t2), axis=1), NEAR, FAR)
    t_max = jnp.clip(jnp.min(jnp.maximum(t1, t2), axis=1), NEAR, FAR)
    ray_norm = jnp.linalg.norm(rays_d, axis=1)
    n_steps = jnp.maximum(
        jnp.ceil((t_max - t_min) * ray_norm / STEPDIST), 1.0)
    start = rays_o + rays_d * t_min[:, None]
    dirn = rays_d / ray_norm[:, None]
    nw = 32
    rpw = N_RAYS // nw
    prep = jnp.stack([
        start[:, 0], start[:, 1], start[:, 2],
        dirn[:, 0], dirn[:, 1], dirn[:, 2],
        n_steps.astype(jnp.float32), jnp.zeros((N_RAYS,), jnp.float32),
    ])
    prep3 = prep.reshape(8, nw, rpw).transpose(1, 0, 2)

    A = jnp.concatenate([density_grid[0], k0_grid[0]], axis=0)
    A = jnp.transpose(A, (1, 2, 3, 0)).reshape(GRID_N, 4)
    table = jnp.concatenate([
        A,
        jnp.roll(A, -1, axis=0),
        jnp.roll(A, -G, axis=0),
        jnp.roll(A, -(G + 1), axis=0),
    ], axis=1)

    return _sc_render(table, prep3)
